# baseline (speedup 1.0000x reference)
import functools

import jax
import jax.numpy as jnp
from jax import lax
from jax.experimental import pallas as pl
from jax.experimental.pallas import tpu as pltpu
from jax.experimental.pallas import tpu_sc as plsc

LANES = 16
TILES = 16
CORES = 2
EBLK = 2048
GB = 64
BAND = 256


@functools.lru_cache(maxsize=None)
def _make_spmm(n, b, nnz_pad):
    e_t = nnz_pad // TILES
    nblocks = nnz_pad // EBLK
    vpr = b // LANES
    acc_w = BAND * b
    slots = nnz_pad + nblocks * GB

    mesh = plsc.VectorSubcoreMesh(core_axis_name="c", subcore_axis_name="s")

    @functools.partial(
        pl.kernel,
        mesh=mesh,
        compiler_params=pltpu.CompilerParams(needs_layout_passes=False),
        out_type=(
            jax.ShapeDtypeStruct((n * b,), jnp.float32),
            jax.ShapeDtypeStruct((CORES * TILES, slots), jnp.int32),
            jax.ShapeDtypeStruct((CORES * TILES, slots), jnp.int32),
            jax.ShapeDtypeStruct((CORES * TILES, slots), jnp.float32),
        ),
        scratch_types=[
            pltpu.VMEM((acc_w,), jnp.float32),
            pltpu.VMEM((EBLK,), jnp.int32),
            pltpu.VMEM((EBLK,), jnp.int32),
            pltpu.VMEM((EBLK,), jnp.float32),
            pltpu.VMEM((EBLK + GB,), jnp.int32),
            pltpu.VMEM((EBLK + GB,), jnp.int32),
            pltpu.VMEM((EBLK + GB,), jnp.float32),
            pltpu.VMEM((EBLK + GB,), jnp.int32),
            pltpu.VMEM((EBLK + GB,), jnp.int32),
            pltpu.VMEM((EBLK + GB,), jnp.float32),
            pltpu.VMEM((GB, b), jnp.float32),
            pltpu.SemaphoreType.DMA,
        ],
    )
    def spmm(zt, rows_h, cols_h, vals_h, out, st_r, st_c, st_v,
             acc, rows_v, cols_v, vals_v,
             r0c, c0c, v0c, r1c, c1c, v1c,
             gbuf, sem):
        cid = lax.axis_index("c")
        sid = lax.axis_index("s")
        wid = cid * TILES + sid

        zvec = jnp.zeros((LANES,), jnp.float32)
        izero = jnp.zeros((LANES,), jnp.int32)
        lane = lax.iota(jnp.int32, LANES)
        band0 = cid * (2 * TILES) + sid
        band1 = band0 + TILES

        def zero_acc():
            def zb(i, _):
                acc[pl.ds(i * LANES, LANES)] = zvec
                return 0
            lax.fori_loop(0, acc_w // LANES, zb, 0)

        def pad_batch(rc, cc, vc, count):
            for p in range(GB // LANES):
                plsc.store_scatter(cc, [count + p * LANES + lane], izero)
                plsc.store_scatter(rc, [count + p * LANES + lane], izero)
                plsc.store_scatter(vc, [count + p * LANES + lane], zvec)

        def process(rc, cc, vc, nbatch):
            def gbody(g, _):
                gg = pl.multiple_of(g * GB, GB)
                pltpu.async_copy(zt.at[cc.at[pl.ds(gg, GB)]], gbuf,
                                 sem).wait()

                def jbody(j, _):
                    idxv = jnp.full((LANES,), gg + j, jnp.int32)
                    rlb = plsc.load_gather(rc, [idxv])
                    vb = plsc.load_gather(vc, [idxv])
                    base = rlb * b + lane
                    for q in range(vpr):
                        g16 = gbuf[j, pl.ds(q * LANES, LANES)]
                        plsc.addupdate_scatter(
                            acc, [base + q * LANES], g16 * vb)
                    return 0
                lax.fori_loop(0, GB, jbody, 0)
                return 0
            lax.fori_loop(0, nbatch, gbody, 0)

        zero_acc()

        def blk_body(blk, spill_n):
            e0 = pl.multiple_of(blk * EBLK, EBLK)
            pltpu.sync_copy(rows_h.at[pl.ds(e0, EBLK)], rows_v)
            pltpu.sync_copy(cols_h.at[pl.ds(e0, EBLK)], cols_v)
            pltpu.sync_copy(vals_h.at[pl.ds(e0, EBLK)], vals_v)

            def cbody(i, counts):
                n0, n1 = counts
                r = rows_v[pl.ds(i * LANES, LANES)]
                c = cols_v[pl.ds(i * LANES, LANES)]
                v = vals_v[pl.ds(i * LANES, LANES)]
                g = lax.shift_right_logical(r, 8)
                rl = r & (BAND - 1)
                m0 = g == band0
                m1 = g == band1
                mi0 = m0.astype(jnp.int32)
                mi1 = m1.astype(jnp.int32)
                p0 = n0 + plsc.cumsum(mi0) - 1
                p1 = n1 + plsc.cumsum(mi1) - 1
                plsc.store_scatter(c0c, [p0], c, mask=m0)
                plsc.store_scatter(r0c, [p0], rl, mask=m0)
                plsc.store_scatter(v0c, [p0], v, mask=m0)
                plsc.store_scatter(c1c, [p1], c, mask=m1)
                plsc.store_scatter(r1c, [p1], rl, mask=m1)
                plsc.store_scatter(v1c, [p1], v, mask=m1)
                return n0 + jnp.sum(mi0), n1 + jnp.sum(mi1)
            n0, n1 = lax.fori_loop(0, EBLK // LANES, cbody,
                                   (jnp.int32(0), jnp.int32(0)))

            pad_batch(r0c, c0c, v0c, n0)
            process(r0c, c0c, v0c, (n0 + GB - 1) // GB)

            pad_batch(r1c, c1c, v1c, n1)
            nsp = (n1 + GB - 1) // GB

            def sbody(k, _):
                src = pl.ds(pl.multiple_of(k * GB, GB), GB)
                dst = pl.ds(pl.multiple_of(spill_n + k * GB, GB), GB)
                pltpu.sync_copy(r1c.at[src], st_r.at[wid, dst])
                pltpu.sync_copy(c1c.at[src], st_c.at[wid, dst])
                pltpu.sync_copy(v1c.at[src], st_v.at[wid, dst])
                return 0
            lax.fori_loop(0, nsp, sbody, 0)
            return spill_n + nsp * GB
        spill_n = lax.fori_loop(0, nblocks, blk_body, jnp.int32(0))

        out0 = pl.multiple_of(band0 * BAND * b, BAND * b)
        pltpu.sync_copy(acc, out.at[pl.ds(out0, acc_w)])

        zero_acc()

        def rblk_body(blk, _):
            s0 = pl.multiple_of(blk * EBLK, EBLK)
            pltpu.sync_copy(st_r.at[wid, pl.ds(s0, EBLK)], rows_v)
            pltpu.sync_copy(st_c.at[wid, pl.ds(s0, EBLK)], cols_v)
            pltpu.sync_copy(st_v.at[wid, pl.ds(s0, EBLK)], vals_v)
            rem = spill_n - s0
            ng = jnp.minimum(rem, EBLK) // GB

            def gbody(g, _):
                gg = pl.multiple_of(g * GB, GB)
                pltpu.async_copy(zt.at[cols_v.at[pl.ds(gg, GB)]], gbuf,
                                 sem).wait()

                def jbody(j, _):
                    idxv = jnp.full((LANES,), gg + j, jnp.int32)
                    rlb = plsc.load_gather(rows_v, [idxv])
                    vb = plsc.load_gather(vals_v, [idxv])
                    base = rlb * b + lane
                    for q in range(vpr):
                        g16 = gbuf[j, pl.ds(q * LANES, LANES)]
                        plsc.addupdate_scatter(
                            acc, [base + q * LANES], g16 * vb)
                    return 0
                lax.fori_loop(0, GB, jbody, 0)
                return 0
            lax.fori_loop(0, ng, gbody, 0)
            return 0
        lax.fori_loop(0, (spill_n + EBLK - 1) // EBLK, rblk_body, 0)

        out1 = pl.multiple_of(band1 * BAND * b, BAND * b)
        pltpu.sync_copy(acc, out.at[pl.ds(out1, acc_w)])

    return spmm


def kernel(Z, A_rows, A_cols, A_vals, logit_alpha):
    b, n = Z.shape
    nnz = A_rows.shape[0]
    nnz_pad = -(-nnz // EBLK) * EBLK
    pad = nnz_pad - nnz
    if pad:
        A_rows = jnp.concatenate(
            [A_rows, jnp.full((pad,), n, jnp.int32)])
        A_cols = jnp.concatenate([A_cols, jnp.zeros((pad,), jnp.int32)])
        A_vals = jnp.concatenate([A_vals, jnp.zeros((pad,), jnp.float32)])
    zt = Z.T
    zs_flat, _, _, _ = _make_spmm(n, b, nnz_pad)(zt, A_rows, A_cols, A_vals)
    zs_t = zs_flat.reshape(n, b)
    alpha = jax.nn.sigmoid(logit_alpha)
    return alpha * zs_t.T + (1.0 - alpha) * Z
